# baseline (speedup 1.0000x reference)
import jax
import jax.numpy as jnp
from jax import lax
from jax.experimental import pallas as pl
from jax.experimental.pallas import tpu as pltpu
from jax.experimental.pallas import tpu_sc as plsc

B = 16384
DIM = 64
N_USERS_ROWS = 100000
N_ENT_ROWS = 1100000
NC = 2
NS = 16
NW = NC * NS
BPW = B // NW
CH = 64
NCH = BPW // CH
LANES = 16

WCOLS = 256
NWIN_E = (1100032 // WCOLS)
NWIN_U = (100096 // WCOLS)
TMAX_E = -(-NWIN_E // NW)
TMAX_U = -(-NWIN_U // NW)
EMAX = 96
NE = 2 * B
NU = B
WSV_E = NWIN_E + 1 + 6
WSV_U = NWIN_U + 1

_LOG1P_COEF = (
    5.621959008883515e-07, 0.999957487075066, -0.49920656854784484,
    0.3269731000138668, -0.22283625832801954, 0.1307650325042385,
    -0.052624851367851076, 0.010119082927824848,
)


def _log1p_poly(t):
    acc = jnp.full_like(t, _LOG1P_COEF[-1])
    for c in reversed(_LOG1P_COEF[:-1]):
        acc = acc * t + jnp.float32(c)
    return acc


def _extract(vref, q, lane):
    base = q & ~(LANES - 1)
    k = q & (LANES - 1)
    vec = vref[pl.ds(base, LANES)]
    return jnp.sum(jnp.where(lane == k, vec, 0), axis=0)


def _phase1_body(es_hbm, us_hbm, wse_hbm, wsu_hbm, etab_t, utab_t,
                 estage, ustage,
                 esv, usv, wsev, wsuv, winbuf, rowbuf, wsem, ssem):
    wid = lax.axis_index("s") * NC + lax.axis_index("c")
    lane = lax.iota(jnp.int32, LANES)

    pltpu.sync_copy(es_hbm, esv)
    pltpu.sync_copy(us_hbm, usv)
    pltpu.sync_copy(wse_hbm, wsev)
    pltpu.sync_copy(wsu_hbm, wsuv)

    def run(tab_t, nwin, tmax, wsv, sidx_v, stage_out):
        def issue(t, slot):
            tg = wid + t * NW

            @pl.when(tg < nwin)
            def _():
                lo = pl.multiple_of(tg * WCOLS, WCOLS)
                pltpu.async_copy(tab_t.at[:, pl.ds(lo, WCOLS)],
                                 winbuf.at[slot], wsem)

        def wait_win(slot):
            pltpu.make_async_copy(tab_t.at[:, pl.ds(0, WCOLS)],
                                  winbuf.at[slot], wsem).wait()

        issue(0, 0)

        def win_body(t, cnt_prev):
            tg = wid + t * NW
            valid = tg < nwin

            @pl.when(valid)
            def _():
                wait_win(t % 2)
            issue(t + 1, (t + 1) % 2)

            def process():
                lo = tg * WCOLS
                jstart = _extract(wsv, tg, lane)
                jend = _extract(wsv, tg + 1, lane)
                slotw = t % 2

                def entry(jj, _):
                    r = _extract(sidx_v, jj, lane)
                    roff = jnp.full((LANES,), r - lo, jnp.int32)
                    rown = jj - jstart
                    for jc in range(DIM // LANES):
                        vals = plsc.load_gather(
                            winbuf.at[slotw],
                            [jc * LANES + lane, roff])
                        rowbuf[slotw, rown, pl.ds(jc * LANES, LANES)] = vals
                    pltpu.async_copy(rowbuf.at[slotw, pl.ds(rown, 1), :],
                                     stage_out.at[pl.ds(jj, 1), :], ssem)
                    return 0

                lax.fori_loop(jstart, jend, entry, 0)
                return jend - jstart

            cnt = lax.cond(valid, process, lambda: 0)

            def dr(k, _):
                pltpu.make_async_copy(stage_out.at[pl.ds(0, 1), :],
                                      rowbuf.at[0, pl.ds(0, 1), :],
                                      ssem).wait()
                return 0
            lax.fori_loop(0, cnt_prev, dr, 0)
            return cnt

        last = lax.fori_loop(0, tmax, win_body, 0)

        def dr(k, _):
            pltpu.make_async_copy(stage_out.at[pl.ds(0, 1), :],
                                  rowbuf.at[0, pl.ds(0, 1), :], ssem).wait()
            return 0
        lax.fori_loop(0, last, dr, 0)

    run(etab_t, NWIN_E, TMAX_E, wsev, esv, estage)
    run(utab_t, NWIN_U, TMAX_U, wsuv, usv, ustage)


def _phase2_body(ur_hbm, pr_hbm, nr_hbm, ustage, estage, out_hbm,
                 uiv, piv, niv, ubuf, pbuf, nbuf, part,
                 usem, psem, nsem):
    wid = lax.axis_index("s") * NC + lax.axis_index("c")
    base = wid * BPW

    pltpu.sync_copy(ur_hbm.at[pl.ds(base, BPW)], uiv)
    pltpu.sync_copy(pr_hbm.at[pl.ds(base, BPW)], piv)
    pltpu.sync_copy(nr_hbm.at[pl.ds(base, BPW)], niv)

    lane = lax.iota(jnp.int32, LANES)

    def fire(c, slot):
        def enq(i, _):
            g = c * CH + (i & ~(LANES - 1))
            k = i & (LANES - 1)
            sel = lane == k
            ru = jnp.sum(jnp.where(sel, uiv[pl.ds(g, LANES)], 0), axis=0)
            rp = jnp.sum(jnp.where(sel, piv[pl.ds(g, LANES)], 0), axis=0)
            rn = jnp.sum(jnp.where(sel, niv[pl.ds(g, LANES)], 0), axis=0)
            pltpu.async_copy(ustage.at[pl.ds(ru, 1), :],
                             ubuf.at[slot, pl.ds(i, 1), :], usem)
            pltpu.async_copy(estage.at[pl.ds(rp, 1), :],
                             pbuf.at[slot, pl.ds(i, 1), :], psem)
            pltpu.async_copy(estage.at[pl.ds(rn, 1), :],
                             nbuf.at[slot, pl.ds(i, 1), :], nsem)
            return 0
        lax.fori_loop(0, CH, enq, 0)

    def drain(slot):
        pltpu.make_async_copy(ustage.at[pl.ds(0, CH), :],
                              ubuf.at[slot], usem).wait()
        pltpu.make_async_copy(estage.at[pl.ds(0, CH), :],
                              pbuf.at[slot], psem).wait()
        pltpu.make_async_copy(estage.at[pl.ds(0, CH), :],
                              nbuf.at[slot], nsem).wait()

    def chunk_sum(slot, acc):
        def row_body(i, a):
            for j in range(DIM // LANES):
                sl = pl.ds(j * LANES, LANES)
                u = ubuf[slot, i, sl]
                p = pbuf[slot, i, sl]
                n = nbuf[slot, i, sl]
                ps = u * p
                ns = u * n
                ea = jnp.exp(-jnp.abs(ps))
                eb = jnp.exp(-jnp.abs(ns))
                a = a + (jnp.maximum(-ps, 0.0) + jnp.maximum(ns, 0.0)
                         + _log1p_poly(ea) + _log1p_poly(eb))
            return a
        return lax.fori_loop(0, CH, row_body, acc)

    acc = jnp.zeros((LANES,), jnp.float32)
    fire(0, 0)
    for c in range(NCH):
        drain(c % 2)
        if c + 1 < NCH:
            fire(c + 1, (c + 1) % 2)
        acc = chunk_sum(c % 2, acc)

    part[...] = acc
    pltpu.sync_copy(part, out_hbm.at[wid])


@jax.jit
def _sc_loss(users, pos, neg, utab, etab):
    mesh = plsc.VectorSubcoreMesh(core_axis_name="c", subcore_axis_name="s")

    eidx = jnp.concatenate([pos, neg])
    eord = jnp.argsort(eidx)
    es = eidx[eord]
    erank = jnp.zeros((NE,), jnp.int32).at[eord].set(
        jnp.arange(NE, dtype=jnp.int32))
    prank, nrank = erank[:B], erank[B:]
    uord = jnp.argsort(users)
    us = users[uord]
    urank = jnp.zeros((NU,), jnp.int32).at[uord].set(
        jnp.arange(NU, dtype=jnp.int32))

    wse = jnp.searchsorted(es, jnp.arange(WSV_E, dtype=jnp.int32) * WCOLS,
                           ).astype(jnp.int32)
    wsu = jnp.searchsorted(us, jnp.arange(WSV_U, dtype=jnp.int32) * WCOLS,
                           ).astype(jnp.int32)
    wsu = jnp.concatenate([wsu, jnp.full((8 - WSV_U % 8,), NU, jnp.int32)])

    es_p = jnp.concatenate([es, jnp.zeros((EMAX + 16,), jnp.int32)])
    us_p = jnp.concatenate([us, jnp.zeros((EMAX + 16,), jnp.int32)])

    phase1 = pl.kernel(
        _phase1_body,
        out_type=(jax.ShapeDtypeStruct((NE, DIM), jnp.float32),
                  jax.ShapeDtypeStruct((NU, DIM), jnp.float32)),
        mesh=mesh,
        compiler_params=pltpu.CompilerParams(needs_layout_passes=False),
        scratch_types=[
            pltpu.VMEM((NE + EMAX + 16,), jnp.int32),
            pltpu.VMEM((NU + EMAX + 16,), jnp.int32),
            pltpu.VMEM((WSV_E,), jnp.int32),
            pltpu.VMEM((WSV_U + 8 - WSV_U % 8,), jnp.int32),
            pltpu.VMEM((2, DIM, WCOLS), jnp.float32),
            pltpu.VMEM((2, EMAX, DIM), jnp.float32),
            pltpu.SemaphoreType.DMA,
            pltpu.SemaphoreType.DMA,
        ],
    )
    estage, ustage = phase1(es_p, us_p, wse, wsu, etab.T, utab.T)

    phase2 = pl.kernel(
        _phase2_body,
        out_type=jax.ShapeDtypeStruct((NW, LANES), jnp.float32),
        mesh=mesh,
        compiler_params=pltpu.CompilerParams(needs_layout_passes=False),
        scratch_types=[
            pltpu.VMEM((BPW,), jnp.int32),
            pltpu.VMEM((BPW,), jnp.int32),
            pltpu.VMEM((BPW,), jnp.int32),
            pltpu.VMEM((2, CH, DIM), jnp.float32),
            pltpu.VMEM((2, CH, DIM), jnp.float32),
            pltpu.VMEM((2, CH, DIM), jnp.float32),
            pltpu.VMEM((LANES,), jnp.float32),
            pltpu.SemaphoreType.DMA,
            pltpu.SemaphoreType.DMA,
            pltpu.SemaphoreType.DMA,
        ],
    )
    parts = phase2(urank, prank, nrank, ustage, estage)
    return jnp.sum(parts) / jnp.float32(B * DIM)


def kernel(users, pos, neg, user_table, entity_table):
    return _sc_loss(users.astype(jnp.int32), pos.astype(jnp.int32),
                    neg.astype(jnp.int32), user_table, entity_table)
